# Initial kernel scaffold; baseline (speedup 1.0000x reference)
import jax
import jax.numpy as jnp
from jax.experimental import pallas as pl


def kernel(x_nchw, a1e, a1o, a2e, a2o, sc1, sh1, sc2, sh2, wfc1, bfc1, wfc2, bfc2):
    raise NotImplementedError("write your pallas kernel here")


# speedup vs baseline: 6.3616x; 6.3616x over previous
import jax
import jax.numpy as jnp
from jax import lax
from jax.experimental import pallas as pl
from jax.experimental.pallas import tpu as pltpu

_BT = 16


def _fused_body(x_ref, a1e_ref, a1o_ref, a2e_ref, a2o_ref,
                sc1_ref, sh1_ref, sc2_ref, sh2_ref,
                p_ref, wfc1_ref, bfc1_ref, wfc2_ref, bfc2_ref, o_ref):
    bt = x_ref.shape[0]
    rows = bt * 8
    bf = jnp.bfloat16

    gs = [x_ref[:, c].reshape(rows, 128).astype(bf) for c in range(3)]
    xm = [jnp.concatenate([g[:, 32 * m:32 * (m + 1)] for g in gs], axis=1)
          for m in range(4)]

    hid = lax.broadcasted_iota(jnp.int32, (rows, 1), 0) % 8
    z96 = jnp.zeros((1, 96), bf)
    sd3 = jnp.where(hid == 0, jnp.zeros((), bf),
                    jnp.concatenate([z96, xm[3][:-1]], axis=0))
    su0 = jnp.where(hid == 7, jnp.zeros((), bf),
                    jnp.concatenate([xm[0][1:], z96], axis=0))

    phases = [
        jnp.concatenate([sd3, xm[0], xm[1]], axis=1),
        jnp.concatenate([xm[0], xm[1], xm[2]], axis=1),
        jnp.concatenate([xm[1], xm[2], xm[3]], axis=1),
        jnp.concatenate([xm[2], xm[3], su0], axis=1),
    ]

    sc1 = sc1_ref[...]
    sh1 = sh1_ref[...]

    def cbr1(xp, w_ref):
        acc = jnp.dot(xp, w_ref[...], preferred_element_type=jnp.float32)
        return jnp.maximum(acc * sc1 + sh1, 0.0)

    y = [jnp.maximum(cbr1(p, a1e_ref), cbr1(p, a1o_ref)) for p in phases]
    y1e = jnp.maximum(y[0], y[1]).astype(bf)
    y1o = jnp.maximum(y[2], y[3]).astype(bf)

    z256 = jnp.zeros((1, 256), bf)
    sdo = jnp.where(hid == 0, jnp.zeros((), bf),
                    jnp.concatenate([z256, y1o[:-1]], axis=0))
    sue = jnp.where(hid == 7, jnp.zeros((), bf),
                    jnp.concatenate([y1e[1:], z256], axis=0))
    yhe = jnp.concatenate([sdo, y1e, y1o], axis=1)
    yho = jnp.concatenate([y1e, y1o, sue], axis=1)

    sc2 = sc2_ref[...]
    sh2 = sh2_ref[...]

    def cbr2(xp, w_ref):
        acc = jnp.dot(xp, w_ref[...], preferred_element_type=jnp.float32)
        return jnp.maximum(acc * sc2 + sh2, 0.0)

    y2 = jnp.maximum(
        jnp.maximum(cbr2(yhe, a2e_ref), cbr2(yhe, a2o_ref)),
        jnp.maximum(cbr2(yho, a2e_ref), cbr2(yho, a2o_ref)))

    y2t = jnp.dot(p_ref[...], y2.astype(bf),
                  preferred_element_type=jnp.float32)
    flat = jnp.concatenate([y2t[r * bt:(r + 1) * bt] for r in range(8)],
                           axis=1)

    h = jnp.maximum(
        jnp.dot(flat, wfc1_ref[...], preferred_element_type=jnp.float32)
        + bfc1_ref[...], 0.0)
    z = jnp.sum(h * wfc2_ref[...], axis=-1, keepdims=True) + bfc2_ref[...]
    o_ref[...] = 1.0 / (1.0 + jnp.exp(-z))


def kernel(x_nchw, a1e, a1o, a2e, a2o, sc1, sh1, sc2, sh2,
           wfc1, bfc1, wfc2, bfc2):
    n = x_nchw.shape[0]
    bt = _BT
    bf = jnp.bfloat16

    x4 = x_nchw.reshape(n, 3, 8, 128)

    a1e_s = a1e.reshape(3, 32, 3, 256).transpose(0, 2, 1, 3).reshape(288, 256)
    a1o_s = a1o.reshape(3, 32, 3, 256).transpose(0, 2, 1, 3).reshape(288, 256)
    a2e_s = a2e.reshape(768, 256)
    a2o_s = a2o.reshape(768, 256)

    rr = jnp.arange(bt * 8)
    perm = ((rr[None, :] == (rr[:, None] % bt) * 8 + rr[:, None] // bt)
            .astype(bf))

    c2 = lambda i: (0, 0)
    out = pl.pallas_call(
        _fused_body,
        out_shape=jax.ShapeDtypeStruct((n, 1), jnp.float32),
        grid=(n // bt,),
        in_specs=[
            pl.BlockSpec((bt, 3, 8, 128), lambda i: (i, 0, 0, 0)),
            pl.BlockSpec((288, 256), c2),
            pl.BlockSpec((288, 256), c2),
            pl.BlockSpec((768, 256), c2),
            pl.BlockSpec((768, 256), c2),
            pl.BlockSpec((1, 256), c2),
            pl.BlockSpec((1, 256), c2),
            pl.BlockSpec((1, 256), c2),
            pl.BlockSpec((1, 256), c2),
            pl.BlockSpec((bt * 8, bt * 8), c2),
            pl.BlockSpec((2048, 64), c2),
            pl.BlockSpec((1, 64), c2),
            pl.BlockSpec((1, 64), c2),
            pl.BlockSpec((1, 1), c2),
        ],
        out_specs=pl.BlockSpec((bt, 1), lambda i: (i, 0)),
        compiler_params=pltpu.CompilerParams(
            dimension_semantics=("parallel",)),
    )(x4, a1e_s.astype(bf), a1o_s.astype(bf),
      a2e_s.astype(bf), a2o_s.astype(bf),
      sc1, sh1, sc2, sh2, perm, wfc1, bfc1, wfc2, bfc2)
    return out


# speedup vs baseline: 8.3616x; 1.3144x over previous
import jax
import jax.numpy as jnp
from jax import lax
from jax.experimental import pallas as pl
from jax.experimental.pallas import tpu as pltpu

_BT = 32


def _fused_body(x_ref, a1e_ref, a1o_ref, a2e_ref, a2o_ref,
                sc1_ref, sh1_ref, sc2_ref, sh2_ref,
                p_ref, wfc1_ref, bfc1_ref, wfc2_ref, bfc2_ref, o_ref):
    bt = x_ref.shape[0]
    rows = bt * 8
    bf = jnp.bfloat16

    gs = [x_ref[:, c].reshape(rows, 128).astype(bf) for c in range(3)]
    xm = [jnp.concatenate([g[:, 32 * m:32 * (m + 1)] for g in gs], axis=1)
          for m in range(4)]

    hid = lax.broadcasted_iota(jnp.int32, (rows, 1), 0) % 8
    z96 = jnp.zeros((1, 96), bf)
    sd3 = jnp.where(hid == 0, jnp.zeros((), bf),
                    jnp.concatenate([z96, xm[3][:-1]], axis=0))
    su0 = jnp.where(hid == 7, jnp.zeros((), bf),
                    jnp.concatenate([xm[0][1:], z96], axis=0))

    phases = [
        jnp.concatenate([sd3, xm[0], xm[1]], axis=1),
        jnp.concatenate([xm[0], xm[1], xm[2]], axis=1),
        jnp.concatenate([xm[1], xm[2], xm[3]], axis=1),
        jnp.concatenate([xm[2], xm[3], su0], axis=1),
    ]

    sc1 = sc1_ref[...]
    sh1 = sh1_ref[...]

    def cbr1(xp, w_ref):
        acc = jnp.dot(xp, w_ref[...], preferred_element_type=jnp.float32)
        return jnp.maximum(acc * sc1 + sh1, 0.0)

    y = [jnp.maximum(cbr1(p, a1e_ref), cbr1(p, a1o_ref)) for p in phases]
    y1e = jnp.maximum(y[0], y[1]).astype(bf)
    y1o = jnp.maximum(y[2], y[3]).astype(bf)

    z256 = jnp.zeros((1, 256), bf)
    sdo = jnp.where(hid == 0, jnp.zeros((), bf),
                    jnp.concatenate([z256, y1o[:-1]], axis=0))
    sue = jnp.where(hid == 7, jnp.zeros((), bf),
                    jnp.concatenate([y1e[1:], z256], axis=0))
    yhe = jnp.concatenate([sdo, y1e, y1o], axis=1)
    yho = jnp.concatenate([y1e, y1o, sue], axis=1)

    sc2 = sc2_ref[...]
    sh2 = sh2_ref[...]

    def cbr2(xp, w_ref):
        acc = jnp.dot(xp, w_ref[...], preferred_element_type=jnp.float32)
        return jnp.maximum(acc * sc2 + sh2, 0.0)

    y2 = jnp.maximum(
        jnp.maximum(cbr2(yhe, a2e_ref), cbr2(yhe, a2o_ref)),
        jnp.maximum(cbr2(yho, a2e_ref), cbr2(yho, a2o_ref)))

    y2t = jnp.dot(p_ref[...], y2.astype(bf),
                  preferred_element_type=jnp.float32)
    flat = jnp.concatenate([y2t[r * bt:(r + 1) * bt] for r in range(8)],
                           axis=1)

    h = jnp.maximum(
        jnp.dot(flat, wfc1_ref[...], preferred_element_type=jnp.float32)
        + bfc1_ref[...], 0.0)
    z = jnp.sum(h * wfc2_ref[...], axis=-1, keepdims=True) + bfc2_ref[...]
    o_ref[...] = 1.0 / (1.0 + jnp.exp(-z))


def kernel(x_nchw, a1e, a1o, a2e, a2o, sc1, sh1, sc2, sh2,
           wfc1, bfc1, wfc2, bfc2):
    n = x_nchw.shape[0]
    bt = _BT
    bf = jnp.bfloat16

    x4 = x_nchw.reshape(n, 3, 8, 128)

    a1e_s = a1e.reshape(3, 32, 3, 256).transpose(0, 2, 1, 3).reshape(288, 256)
    a1o_s = a1o.reshape(3, 32, 3, 256).transpose(0, 2, 1, 3).reshape(288, 256)
    a2e_s = a2e.reshape(768, 256)
    a2o_s = a2o.reshape(768, 256)

    rr = jnp.arange(bt * 8)
    perm = ((rr[None, :] == (rr[:, None] % bt) * 8 + rr[:, None] // bt)
            .astype(bf))

    c2 = lambda i: (0, 0)
    out = pl.pallas_call(
        _fused_body,
        out_shape=jax.ShapeDtypeStruct((n, 1), jnp.float32),
        grid=(n // bt,),
        in_specs=[
            pl.BlockSpec((bt, 3, 8, 128), lambda i: (i, 0, 0, 0)),
            pl.BlockSpec((288, 256), c2),
            pl.BlockSpec((288, 256), c2),
            pl.BlockSpec((768, 256), c2),
            pl.BlockSpec((768, 256), c2),
            pl.BlockSpec((1, 256), c2),
            pl.BlockSpec((1, 256), c2),
            pl.BlockSpec((1, 256), c2),
            pl.BlockSpec((1, 256), c2),
            pl.BlockSpec((bt * 8, bt * 8), c2),
            pl.BlockSpec((2048, 64), c2),
            pl.BlockSpec((1, 64), c2),
            pl.BlockSpec((1, 64), c2),
            pl.BlockSpec((1, 1), c2),
        ],
        out_specs=pl.BlockSpec((bt, 1), lambda i: (i, 0)),
        compiler_params=pltpu.CompilerParams(
            dimension_semantics=("parallel",)),
    )(x4, a1e_s.astype(bf), a1o_s.astype(bf),
      a2e_s.astype(bf), a2o_s.astype(bf),
      sc1, sh1, sc2, sh2, perm, wfc1, bfc1, wfc2, bfc2)
    return out


# speedup vs baseline: 10.0397x; 1.2007x over previous
import jax
import jax.numpy as jnp
from jax import lax
from jax.experimental import pallas as pl
from jax.experimental.pallas import tpu as pltpu

_BT = 64


def _fused_body(x_ref, a1e_ref, a1o_ref, a2e_ref, a2o_ref,
                sc1_ref, sh1_ref, sc2_ref, sh2_ref,
                p_ref, wfc1_ref, bfc1_ref, wfc2_ref, bfc2_ref, o_ref):
    bt = x_ref.shape[0]
    rows = bt * 8
    bf = jnp.bfloat16

    gs = [x_ref[:, c].reshape(rows, 128).astype(bf) for c in range(3)]
    xm = [jnp.concatenate([g[:, 32 * m:32 * (m + 1)] for g in gs], axis=1)
          for m in range(4)]

    hid = lax.broadcasted_iota(jnp.int32, (rows, 1), 0) % 8
    z96 = jnp.zeros((1, 96), bf)
    sd3 = jnp.where(hid == 0, jnp.zeros((), bf),
                    jnp.concatenate([z96, xm[3][:-1]], axis=0))
    su0 = jnp.where(hid == 7, jnp.zeros((), bf),
                    jnp.concatenate([xm[0][1:], z96], axis=0))

    phases = [
        jnp.concatenate([sd3, xm[0], xm[1]], axis=1),
        jnp.concatenate([xm[0], xm[1], xm[2]], axis=1),
        jnp.concatenate([xm[1], xm[2], xm[3]], axis=1),
        jnp.concatenate([xm[2], xm[3], su0], axis=1),
    ]

    sc1 = sc1_ref[...]
    sh1 = sh1_ref[...]

    def cbr1(xp, w_ref):
        acc = jnp.dot(xp, w_ref[...], preferred_element_type=jnp.float32)
        return jnp.maximum(acc * sc1 + sh1, 0.0)

    y = [jnp.maximum(cbr1(p, a1e_ref), cbr1(p, a1o_ref)) for p in phases]
    y1e = jnp.maximum(y[0], y[1]).astype(bf)
    y1o = jnp.maximum(y[2], y[3]).astype(bf)

    z256 = jnp.zeros((1, 256), bf)
    sdo = jnp.where(hid == 0, jnp.zeros((), bf),
                    jnp.concatenate([z256, y1o[:-1]], axis=0))
    sue = jnp.where(hid == 7, jnp.zeros((), bf),
                    jnp.concatenate([y1e[1:], z256], axis=0))
    yhe = jnp.concatenate([sdo, y1e, y1o], axis=1)
    yho = jnp.concatenate([y1e, y1o, sue], axis=1)

    sc2 = sc2_ref[...]
    sh2 = sh2_ref[...]

    def cbr2(xp, w_ref):
        acc = jnp.dot(xp, w_ref[...], preferred_element_type=jnp.float32)
        return jnp.maximum(acc * sc2 + sh2, 0.0)

    y2 = jnp.maximum(
        jnp.maximum(cbr2(yhe, a2e_ref), cbr2(yhe, a2o_ref)),
        jnp.maximum(cbr2(yho, a2e_ref), cbr2(yho, a2o_ref)))

    y2t = jnp.dot(p_ref[...], y2.astype(bf),
                  preferred_element_type=jnp.float32)
    flat = jnp.concatenate([y2t[r * bt:(r + 1) * bt] for r in range(8)],
                           axis=1)

    h = jnp.maximum(
        jnp.dot(flat, wfc1_ref[...], preferred_element_type=jnp.float32)
        + bfc1_ref[...], 0.0)
    z = jnp.sum(h * wfc2_ref[...], axis=-1, keepdims=True) + bfc2_ref[...]
    o_ref[...] = 1.0 / (1.0 + jnp.exp(-z))


def kernel(x_nchw, a1e, a1o, a2e, a2o, sc1, sh1, sc2, sh2,
           wfc1, bfc1, wfc2, bfc2):
    n = x_nchw.shape[0]
    bt = _BT
    bf = jnp.bfloat16

    x4 = x_nchw.reshape(n, 3, 8, 128)

    a1e_s = a1e.reshape(3, 32, 3, 256).transpose(0, 2, 1, 3).reshape(288, 256)
    a1o_s = a1o.reshape(3, 32, 3, 256).transpose(0, 2, 1, 3).reshape(288, 256)
    a2e_s = a2e.reshape(768, 256)
    a2o_s = a2o.reshape(768, 256)

    rr = jnp.arange(bt * 8)
    perm = ((rr[None, :] == (rr[:, None] % bt) * 8 + rr[:, None] // bt)
            .astype(bf))

    c2 = lambda i: (0, 0)
    out = pl.pallas_call(
        _fused_body,
        out_shape=jax.ShapeDtypeStruct((n, 1), jnp.float32),
        grid=(n // bt,),
        in_specs=[
            pl.BlockSpec((bt, 3, 8, 128), lambda i: (i, 0, 0, 0)),
            pl.BlockSpec((288, 256), c2),
            pl.BlockSpec((288, 256), c2),
            pl.BlockSpec((768, 256), c2),
            pl.BlockSpec((768, 256), c2),
            pl.BlockSpec((1, 256), c2),
            pl.BlockSpec((1, 256), c2),
            pl.BlockSpec((1, 256), c2),
            pl.BlockSpec((1, 256), c2),
            pl.BlockSpec((bt * 8, bt * 8), c2),
            pl.BlockSpec((2048, 64), c2),
            pl.BlockSpec((1, 64), c2),
            pl.BlockSpec((1, 64), c2),
            pl.BlockSpec((1, 1), c2),
        ],
        out_specs=pl.BlockSpec((bt, 1), lambda i: (i, 0)),
        compiler_params=pltpu.CompilerParams(
            dimension_semantics=("parallel",)),
    )(x4, a1e_s.astype(bf), a1o_s.astype(bf),
      a2e_s.astype(bf), a2o_s.astype(bf),
      sc1, sh1, sc2, sh2, perm, wfc1, bfc1, wfc2, bfc2)
    return out


# speedup vs baseline: 10.1542x; 1.0114x over previous
import jax
import jax.numpy as jnp
from jax import lax
from jax.experimental import pallas as pl
from jax.experimental.pallas import tpu as pltpu

_BT = 128


def _fused_body(x_ref, a1e_ref, a1o_ref, a2e_ref, a2o_ref,
                sc1_ref, sh1_ref, sc2_ref, sh2_ref,
                p_ref, wfc1_ref, bfc1_ref, wfc2_ref, bfc2_ref, o_ref):
    bt = x_ref.shape[0]
    rows = bt * 8
    bf = jnp.bfloat16

    gs = [x_ref[:, c].reshape(rows, 128).astype(bf) for c in range(3)]
    xm = [jnp.concatenate([g[:, 32 * m:32 * (m + 1)] for g in gs], axis=1)
          for m in range(4)]

    hid = lax.broadcasted_iota(jnp.int32, (rows, 1), 0) % 8
    z96 = jnp.zeros((1, 96), bf)
    sd3 = jnp.where(hid == 0, jnp.zeros((), bf),
                    jnp.concatenate([z96, xm[3][:-1]], axis=0))
    su0 = jnp.where(hid == 7, jnp.zeros((), bf),
                    jnp.concatenate([xm[0][1:], z96], axis=0))

    phases = [
        jnp.concatenate([sd3, xm[0], xm[1]], axis=1),
        jnp.concatenate([xm[0], xm[1], xm[2]], axis=1),
        jnp.concatenate([xm[1], xm[2], xm[3]], axis=1),
        jnp.concatenate([xm[2], xm[3], su0], axis=1),
    ]

    sc1 = sc1_ref[...]
    sh1 = sh1_ref[...]

    def cbr1(xp, w_ref):
        acc = jnp.dot(xp, w_ref[...], preferred_element_type=jnp.float32)
        return jnp.maximum(acc * sc1 + sh1, 0.0)

    y = [jnp.maximum(cbr1(p, a1e_ref), cbr1(p, a1o_ref)) for p in phases]
    y1e = jnp.maximum(y[0], y[1]).astype(bf)
    y1o = jnp.maximum(y[2], y[3]).astype(bf)

    z256 = jnp.zeros((1, 256), bf)
    sdo = jnp.where(hid == 0, jnp.zeros((), bf),
                    jnp.concatenate([z256, y1o[:-1]], axis=0))
    sue = jnp.where(hid == 7, jnp.zeros((), bf),
                    jnp.concatenate([y1e[1:], z256], axis=0))
    yhe = jnp.concatenate([sdo, y1e, y1o], axis=1)
    yho = jnp.concatenate([y1e, y1o, sue], axis=1)

    sc2 = sc2_ref[...]
    sh2 = sh2_ref[...]

    def cbr2(xp, w_ref):
        acc = jnp.dot(xp, w_ref[...], preferred_element_type=jnp.float32)
        return jnp.maximum(acc * sc2 + sh2, 0.0)

    y2 = jnp.maximum(
        jnp.maximum(cbr2(yhe, a2e_ref), cbr2(yhe, a2o_ref)),
        jnp.maximum(cbr2(yho, a2e_ref), cbr2(yho, a2o_ref)))

    y2t = jnp.dot(p_ref[...], y2.astype(bf),
                  preferred_element_type=jnp.float32)
    flat = jnp.concatenate([y2t[r * bt:(r + 1) * bt] for r in range(8)],
                           axis=1)

    h = jnp.maximum(
        jnp.dot(flat, wfc1_ref[...], preferred_element_type=jnp.float32)
        + bfc1_ref[...], 0.0)
    z = jnp.sum(h * wfc2_ref[...], axis=-1, keepdims=True) + bfc2_ref[...]
    o_ref[...] = 1.0 / (1.0 + jnp.exp(-z))


def kernel(x_nchw, a1e, a1o, a2e, a2o, sc1, sh1, sc2, sh2,
           wfc1, bfc1, wfc2, bfc2):
    n = x_nchw.shape[0]
    bt = _BT
    bf = jnp.bfloat16

    x4 = x_nchw.reshape(n, 3, 8, 128)

    a1e_s = a1e.reshape(3, 32, 3, 256).transpose(0, 2, 1, 3).reshape(288, 256)
    a1o_s = a1o.reshape(3, 32, 3, 256).transpose(0, 2, 1, 3).reshape(288, 256)
    a2e_s = a2e.reshape(768, 256)
    a2o_s = a2o.reshape(768, 256)

    rr = jnp.arange(bt * 8)
    perm = ((rr[None, :] == (rr[:, None] % bt) * 8 + rr[:, None] // bt)
            .astype(bf))

    c2 = lambda i: (0, 0)
    out = pl.pallas_call(
        _fused_body,
        out_shape=jax.ShapeDtypeStruct((n, 1), jnp.float32),
        grid=(n // bt,),
        in_specs=[
            pl.BlockSpec((bt, 3, 8, 128), lambda i: (i, 0, 0, 0)),
            pl.BlockSpec((288, 256), c2),
            pl.BlockSpec((288, 256), c2),
            pl.BlockSpec((768, 256), c2),
            pl.BlockSpec((768, 256), c2),
            pl.BlockSpec((1, 256), c2),
            pl.BlockSpec((1, 256), c2),
            pl.BlockSpec((1, 256), c2),
            pl.BlockSpec((1, 256), c2),
            pl.BlockSpec((bt * 8, bt * 8), c2),
            pl.BlockSpec((2048, 64), c2),
            pl.BlockSpec((1, 64), c2),
            pl.BlockSpec((1, 64), c2),
            pl.BlockSpec((1, 1), c2),
        ],
        out_specs=pl.BlockSpec((bt, 1), lambda i: (i, 0)),
        compiler_params=pltpu.CompilerParams(
            dimension_semantics=("parallel",)),
    )(x4, a1e_s.astype(bf), a1o_s.astype(bf),
      a2e_s.astype(bf), a2o_s.astype(bf),
      sc1, sh1, sc2, sh2, perm, wfc1, bfc1, wfc2, bfc2)
    return out


# speedup vs baseline: 10.6393x; 1.0478x over previous
import jax
import jax.numpy as jnp
from jax.experimental import pallas as pl
from jax.experimental.pallas import tpu as pltpu

_BT = 128


def _fused_body(x_ref, a1_ref, a2_ref, sc1_ref, sh1_ref, sc2_ref, sh2_ref,
                wfc1_ref, bfc1_ref, wfc2_ref, bfc2_ref, o_ref):
    bt = x_ref.shape[0]
    bf = jnp.bfloat16

    gc = [x_ref[:, c].astype(bf) for c in range(3)]
    xh = [jnp.concatenate(
        [gc[c][:, h // 4, 32 * (h % 4):32 * (h % 4) + 32] for c in range(3)],
        axis=1) for h in range(32)]

    sc1 = sc1_ref[...]
    sh1 = sh1_ref[...]
    sc2 = sc2_ref[...]
    sh2 = sh2_ref[...]

    def conv1_row(h):
        lo, hi = max(h - 1, 0), min(h + 1, 31)
        x3 = (xh[lo] if lo == hi else
              jnp.concatenate(xh[lo:hi + 1], axis=1))
        k0, k1 = (lo - h + 1) * 96, (hi - h + 2) * 96
        acc_e = jnp.dot(x3, a1_ref[k0:k1, 0:256],
                        preferred_element_type=jnp.float32)
        acc_o = jnp.dot(x3, a1_ref[k0:k1, 256:512],
                        preferred_element_type=jnp.float32)
        return (jnp.maximum(acc_e * sc1 + sh1, 0.0),
                jnp.maximum(acc_o * sc1 + sh1, 0.0))

    y1 = []
    for k in range(16):
        e0, o0 = conv1_row(2 * k)
        e1, o1 = conv1_row(2 * k + 1)
        y1.append(jnp.maximum(jnp.maximum(e0, o0),
                              jnp.maximum(e1, o1)).astype(bf))

    def conv2_row(h):
        acc_e = None
        acc_o = None
        for di in range(3):
            src = h + di - 1
            if src < 0 or src > 15:
                continue
            e = jnp.dot(y1[src], a2_ref[256 * di:256 * di + 256, 0:256],
                        preferred_element_type=jnp.float32)
            o = jnp.dot(y1[src], a2_ref[256 * di:256 * di + 256, 256:512],
                        preferred_element_type=jnp.float32)
            acc_e = e if acc_e is None else acc_e + e
            acc_o = o if acc_o is None else acc_o + o
        return (jnp.maximum(acc_e * sc2 + sh2, 0.0),
                jnp.maximum(acc_o * sc2 + sh2, 0.0))

    y2 = []
    for r in range(8):
        e0, o0 = conv2_row(2 * r)
        e1, o1 = conv2_row(2 * r + 1)
        y2.append(jnp.maximum(jnp.maximum(e0, o0), jnp.maximum(e1, o1)))

    flat = jnp.concatenate(y2, axis=1)

    h = jnp.maximum(
        jnp.dot(flat, wfc1_ref[...], preferred_element_type=jnp.float32)
        + bfc1_ref[...], 0.0)
    z = jnp.sum(h * wfc2_ref[...], axis=-1, keepdims=True) + bfc2_ref[...]
    o_ref[...] = 1.0 / (1.0 + jnp.exp(-z))


def kernel(x_nchw, a1e, a1o, a2e, a2o, sc1, sh1, sc2, sh2,
           wfc1, bfc1, wfc2, bfc2):
    n = x_nchw.shape[0]
    bt = min(_BT, n)
    bf = jnp.bfloat16

    x4 = x_nchw.reshape(n, 3, 8, 128)

    def stack1(a):
        return a.reshape(3, 32, 3, 256).transpose(0, 2, 1, 3).reshape(288, 256)

    a1 = jnp.concatenate([stack1(a1e), stack1(a1o)], axis=1).astype(bf)
    a2 = jnp.concatenate([a2e.reshape(768, 256), a2o.reshape(768, 256)],
                         axis=1).astype(bf)

    c2 = lambda i: (0, 0)
    out = pl.pallas_call(
        _fused_body,
        out_shape=jax.ShapeDtypeStruct((n, 1), jnp.float32),
        grid=(n // bt,),
        in_specs=[
            pl.BlockSpec((bt, 3, 8, 128), lambda i: (i, 0, 0, 0)),
            pl.BlockSpec((288, 512), c2),
            pl.BlockSpec((768, 512), c2),
            pl.BlockSpec((1, 256), c2),
            pl.BlockSpec((1, 256), c2),
            pl.BlockSpec((1, 256), c2),
            pl.BlockSpec((1, 256), c2),
            pl.BlockSpec((2048, 64), c2),
            pl.BlockSpec((1, 64), c2),
            pl.BlockSpec((1, 64), c2),
            pl.BlockSpec((1, 1), c2),
        ],
        out_specs=pl.BlockSpec((bt, 1), lambda i: (i, 0)),
        compiler_params=pltpu.CompilerParams(
            dimension_semantics=("parallel",)),
    )(x4, a1, a2, sc1, sh1, sc2, sh2, wfc1, bfc1, wfc2, bfc2)
    return out


# speedup vs baseline: 11.3978x; 1.0713x over previous
import jax
import jax.numpy as jnp
from jax.experimental import pallas as pl
from jax.experimental.pallas import tpu as pltpu

_BT = 256


def _fused_body(x_ref, a1_ref, a2_ref, sc1_ref, sh1_ref, sc2_ref, sh2_ref,
                wfc1_ref, bfc1_ref, wfc2_ref, bfc2_ref, o_ref):
    bt = x_ref.shape[0]
    bf = jnp.bfloat16

    gc = [x_ref[:, c].astype(bf) for c in range(3)]
    xh = [jnp.concatenate(
        [gc[c][:, h // 4, 32 * (h % 4):32 * (h % 4) + 32] for c in range(3)],
        axis=1) for h in range(32)]

    sc1 = sc1_ref[...]
    sh1 = sh1_ref[...]
    sc2 = sc2_ref[...]
    sh2 = sh2_ref[...]

    def conv1_row(h):
        lo, hi = max(h - 1, 0), min(h + 1, 31)
        x3 = (xh[lo] if lo == hi else
              jnp.concatenate(xh[lo:hi + 1], axis=1))
        k0, k1 = (lo - h + 1) * 96, (hi - h + 2) * 96
        acc_e = jnp.dot(x3, a1_ref[k0:k1, 0:256],
                        preferred_element_type=jnp.float32)
        acc_o = jnp.dot(x3, a1_ref[k0:k1, 256:512],
                        preferred_element_type=jnp.float32)
        return (jnp.maximum(acc_e * sc1 + sh1, 0.0),
                jnp.maximum(acc_o * sc1 + sh1, 0.0))

    y1 = []
    for k in range(16):
        e0, o0 = conv1_row(2 * k)
        e1, o1 = conv1_row(2 * k + 1)
        y1.append(jnp.maximum(jnp.maximum(e0, o0),
                              jnp.maximum(e1, o1)).astype(bf))

    def conv2_row(h):
        acc_e = None
        acc_o = None
        for di in range(3):
            src = h + di - 1
            if src < 0 or src > 15:
                continue
            e = jnp.dot(y1[src], a2_ref[256 * di:256 * di + 256, 0:256],
                        preferred_element_type=jnp.float32)
            o = jnp.dot(y1[src], a2_ref[256 * di:256 * di + 256, 256:512],
                        preferred_element_type=jnp.float32)
            acc_e = e if acc_e is None else acc_e + e
            acc_o = o if acc_o is None else acc_o + o
        return (jnp.maximum(acc_e * sc2 + sh2, 0.0),
                jnp.maximum(acc_o * sc2 + sh2, 0.0))

    y2 = []
    for r in range(8):
        e0, o0 = conv2_row(2 * r)
        e1, o1 = conv2_row(2 * r + 1)
        y2.append(jnp.maximum(jnp.maximum(e0, o0), jnp.maximum(e1, o1)))

    flat = jnp.concatenate(y2, axis=1)

    h = jnp.maximum(
        jnp.dot(flat, wfc1_ref[...], preferred_element_type=jnp.float32)
        + bfc1_ref[...], 0.0)
    z = jnp.sum(h * wfc2_ref[...], axis=-1, keepdims=True) + bfc2_ref[...]
    o_ref[...] = 1.0 / (1.0 + jnp.exp(-z))


def kernel(x_nchw, a1e, a1o, a2e, a2o, sc1, sh1, sc2, sh2,
           wfc1, bfc1, wfc2, bfc2):
    n = x_nchw.shape[0]
    bt = min(_BT, n)
    bf = jnp.bfloat16

    x4 = x_nchw.reshape(n, 3, 8, 128)

    def stack1(a):
        return a.reshape(3, 32, 3, 256).transpose(0, 2, 1, 3).reshape(288, 256)

    a1 = jnp.concatenate([stack1(a1e), stack1(a1o)], axis=1).astype(bf)
    a2 = jnp.concatenate([a2e.reshape(768, 256), a2o.reshape(768, 256)],
                         axis=1).astype(bf)

    c2 = lambda i: (0, 0)
    out = pl.pallas_call(
        _fused_body,
        out_shape=jax.ShapeDtypeStruct((n, 1), jnp.float32),
        grid=(n // bt,),
        in_specs=[
            pl.BlockSpec((bt, 3, 8, 128), lambda i: (i, 0, 0, 0)),
            pl.BlockSpec((288, 512), c2),
            pl.BlockSpec((768, 512), c2),
            pl.BlockSpec((1, 256), c2),
            pl.BlockSpec((1, 256), c2),
            pl.BlockSpec((1, 256), c2),
            pl.BlockSpec((1, 256), c2),
            pl.BlockSpec((2048, 64), c2),
            pl.BlockSpec((1, 64), c2),
            pl.BlockSpec((1, 64), c2),
            pl.BlockSpec((1, 1), c2),
        ],
        out_specs=pl.BlockSpec((bt, 1), lambda i: (i, 0)),
        compiler_params=pltpu.CompilerParams(
            dimension_semantics=("parallel",)),
    )(x4, a1, a2, sc1, sh1, sc2, sh2, wfc1, bfc1, wfc2, bfc2)
    return out


# speedup vs baseline: 12.2650x; 1.0761x over previous
import jax
import jax.numpy as jnp
from jax.experimental import pallas as pl
from jax.experimental.pallas import tpu as pltpu

_BT = 512


def _fused_body(x_ref, a1_ref, a2_ref, sc1_ref, sh1_ref, sc2_ref, sh2_ref,
                wfc1_ref, bfc1_ref, wfc2_ref, bfc2_ref, o_ref):
    bt = x_ref.shape[0]
    bf = jnp.bfloat16

    gc = [x_ref[:, c].astype(bf) for c in range(3)]
    xh = [jnp.concatenate(
        [gc[c][:, h // 4, 32 * (h % 4):32 * (h % 4) + 32] for c in range(3)],
        axis=1) for h in range(32)]

    sc1 = sc1_ref[...]
    sh1 = sh1_ref[...]
    sc2 = sc2_ref[...]
    sh2 = sh2_ref[...]

    def conv1_row(h):
        lo, hi = max(h - 1, 0), min(h + 1, 31)
        x3 = (xh[lo] if lo == hi else
              jnp.concatenate(xh[lo:hi + 1], axis=1))
        k0, k1 = (lo - h + 1) * 96, (hi - h + 2) * 96
        acc_e = jnp.dot(x3, a1_ref[k0:k1, 0:256],
                        preferred_element_type=jnp.float32)
        acc_o = jnp.dot(x3, a1_ref[k0:k1, 256:512],
                        preferred_element_type=jnp.float32)
        return (jnp.maximum(acc_e * sc1 + sh1, 0.0),
                jnp.maximum(acc_o * sc1 + sh1, 0.0))

    y1 = []
    for k in range(16):
        e0, o0 = conv1_row(2 * k)
        e1, o1 = conv1_row(2 * k + 1)
        y1.append(jnp.maximum(jnp.maximum(e0, o0),
                              jnp.maximum(e1, o1)).astype(bf))

    def conv2_row(h):
        acc_e = None
        acc_o = None
        for di in range(3):
            src = h + di - 1
            if src < 0 or src > 15:
                continue
            e = jnp.dot(y1[src], a2_ref[256 * di:256 * di + 256, 0:256],
                        preferred_element_type=jnp.float32)
            o = jnp.dot(y1[src], a2_ref[256 * di:256 * di + 256, 256:512],
                        preferred_element_type=jnp.float32)
            acc_e = e if acc_e is None else acc_e + e
            acc_o = o if acc_o is None else acc_o + o
        return (jnp.maximum(acc_e * sc2 + sh2, 0.0),
                jnp.maximum(acc_o * sc2 + sh2, 0.0))

    y2 = []
    for r in range(8):
        e0, o0 = conv2_row(2 * r)
        e1, o1 = conv2_row(2 * r + 1)
        y2.append(jnp.maximum(jnp.maximum(e0, o0), jnp.maximum(e1, o1)))

    flat = jnp.concatenate(y2, axis=1)

    h = jnp.maximum(
        jnp.dot(flat, wfc1_ref[...], preferred_element_type=jnp.float32)
        + bfc1_ref[...], 0.0)
    z = jnp.sum(h * wfc2_ref[...], axis=-1, keepdims=True) + bfc2_ref[...]
    o_ref[...] = 1.0 / (1.0 + jnp.exp(-z))


def kernel(x_nchw, a1e, a1o, a2e, a2o, sc1, sh1, sc2, sh2,
           wfc1, bfc1, wfc2, bfc2):
    n = x_nchw.shape[0]
    bt = min(_BT, n)
    bf = jnp.bfloat16

    x4 = x_nchw.reshape(n, 3, 8, 128)

    def stack1(a):
        return a.reshape(3, 32, 3, 256).transpose(0, 2, 1, 3).reshape(288, 256)

    a1 = jnp.concatenate([stack1(a1e), stack1(a1o)], axis=1).astype(bf)
    a2 = jnp.concatenate([a2e.reshape(768, 256), a2o.reshape(768, 256)],
                         axis=1).astype(bf)

    c2 = lambda i: (0, 0)
    out = pl.pallas_call(
        _fused_body,
        out_shape=jax.ShapeDtypeStruct((n, 1), jnp.float32),
        grid=(n // bt,),
        in_specs=[
            pl.BlockSpec((bt, 3, 8, 128), lambda i: (i, 0, 0, 0)),
            pl.BlockSpec((288, 512), c2),
            pl.BlockSpec((768, 512), c2),
            pl.BlockSpec((1, 256), c2),
            pl.BlockSpec((1, 256), c2),
            pl.BlockSpec((1, 256), c2),
            pl.BlockSpec((1, 256), c2),
            pl.BlockSpec((2048, 64), c2),
            pl.BlockSpec((1, 64), c2),
            pl.BlockSpec((1, 64), c2),
            pl.BlockSpec((1, 1), c2),
        ],
        out_specs=pl.BlockSpec((bt, 1), lambda i: (i, 0)),
        compiler_params=pltpu.CompilerParams(
            dimension_semantics=("parallel",)),
    )(x4, a1, a2, sc1, sh1, sc2, sh2, wfc1, bfc1, wfc2, bfc2)
    return out
